# baseline (speedup 1.0000x reference)
import functools

import jax
import jax.numpy as jnp
from jax import lax
from jax.experimental import pallas as pl
from jax.experimental.pallas import tpu as pltpu
from jax.experimental.pallas import tpu_sc as plsc


@functools.lru_cache(maxsize=None)
def _build_gather(B, D, chunk):
    info = plsc.get_sparse_core_info()
    NC, NS = info.num_cores, info.num_subcores
    NW = NC * NS
    assert B % (NW * chunk) == 0
    b_per_w = B // NW
    n_chunks = b_per_w // chunk
    mesh = plsc.VectorSubcoreMesh(core_axis_name="c", subcore_axis_name="s")

    nbuf = 12

    @functools.partial(
        pl.kernel,
        mesh=mesh,
        out_type=jax.ShapeDtypeStruct((B, D), jnp.float32),
        scratch_types=[
            pltpu.VMEM((n_chunks, chunk), jnp.int32),
        ]
        + [pltpu.VMEM((chunk, D), jnp.float32) for _ in range(nbuf)]
        + [pltpu.SemaphoreType.DMA for _ in range(2 * nbuf)],
    )
    def k(idx_hbm, table_hbm, out_hbm, idx_v, *rest):
        bufs = rest[:nbuf]
        gsems = rest[nbuf : 2 * nbuf]
        ssems = rest[2 * nbuf :]
        wid = lax.axis_index("s") * NC + lax.axis_index("c")
        pltpu.sync_copy(idx_hbm.at[wid], idx_v)
        gcp = [None] * nbuf
        scp = [None] * nbuf
        for step in range(n_chunks + nbuf - 1):
            slot = step % nbuf
            if step < n_chunks:
                if step >= nbuf:
                    scp[slot].wait()
                gcp[slot] = pltpu.async_copy(
                    table_hbm.at[idx_v.at[step]], bufs[slot], gsems[slot]
                )
            g = step - (nbuf - 1)
            if g >= 0:
                gslot = g % nbuf
                gcp[gslot].wait()
                scp[gslot] = pltpu.async_copy(
                    bufs[gslot],
                    out_hbm.at[pl.ds((g * NW + wid) * chunk, chunk)],
                    ssems[gslot],
                )
        for g in range(max(0, n_chunks - nbuf), n_chunks):
            scp[g % nbuf].wait()

    return k, NW, n_chunks, chunk


def kernel(positions, pe):
    Bb, S = positions.shape
    V, D = pe.shape
    B = Bb * S
    chunk = 8
    k, NW, n_chunks, chunk = _build_gather(B, D, chunk)
    idx = (
        positions.reshape(n_chunks, NW, chunk).transpose(1, 0, 2).astype(jnp.int32)
    )
    out = k(idx, pe)
    return out.reshape(Bb, S, D)


# speedup vs baseline: 1.0195x; 1.0195x over previous
import functools

import jax
import jax.numpy as jnp
from jax import lax
from jax.experimental import pallas as pl
from jax.experimental.pallas import tpu as pltpu
from jax.experimental.pallas import tpu_sc as plsc


@functools.lru_cache(maxsize=None)
def _build_gather(B, D, chunk):
    info = plsc.get_sparse_core_info()
    NC, NS = info.num_cores, info.num_subcores
    NW = NC * NS
    assert B % (NW * chunk) == 0
    b_per_w = B // NW
    n_chunks = b_per_w // chunk
    mesh = plsc.VectorSubcoreMesh(core_axis_name="c", subcore_axis_name="s")

    nbuf = 3

    @functools.partial(
        pl.kernel,
        mesh=mesh,
        out_type=jax.ShapeDtypeStruct((B, D), jnp.float32),
        scratch_types=[
            pltpu.VMEM((n_chunks, chunk), jnp.int32),
        ]
        + [pltpu.VMEM((chunk, D), jnp.float32) for _ in range(nbuf)]
        + [pltpu.SemaphoreType.DMA for _ in range(2 * nbuf)],
    )
    def k(idx_hbm, table_hbm, out_hbm, idx_v, *rest):
        bufs = rest[:nbuf]
        gsems = rest[nbuf : 2 * nbuf]
        ssems = rest[2 * nbuf :]
        wid = lax.axis_index("s") * NC + lax.axis_index("c")
        pltpu.sync_copy(idx_hbm.at[wid], idx_v)
        gcp = [None] * nbuf
        scp = [None] * nbuf
        for step in range(n_chunks + nbuf - 1):
            slot = step % nbuf
            if step < n_chunks:
                if step >= nbuf:
                    scp[slot].wait()
                gcp[slot] = pltpu.async_copy(
                    table_hbm.at[idx_v.at[step]], bufs[slot], gsems[slot]
                )
            g = step - (nbuf - 1)
            if g >= 0:
                gslot = g % nbuf
                gcp[gslot].wait()
                scp[gslot] = pltpu.async_copy(
                    bufs[gslot],
                    out_hbm.at[pl.ds((g * NW + wid) * chunk, chunk)],
                    ssems[gslot],
                )
        for g in range(max(0, n_chunks - nbuf), n_chunks):
            scp[g % nbuf].wait()

    return k, NW, n_chunks, chunk


def kernel(positions, pe):
    Bb, S = positions.shape
    V, D = pe.shape
    B = Bb * S
    chunk = 32
    k, NW, n_chunks, chunk = _build_gather(B, D, chunk)
    idx = (
        positions.reshape(n_chunks, NW, chunk).transpose(1, 0, 2).astype(jnp.int32)
    )
    out = k(idx, pe)
    return out.reshape(Bb, S, D)


# speedup vs baseline: 1.0256x; 1.0060x over previous
import functools

import jax
import jax.numpy as jnp
from jax import lax
from jax.experimental import pallas as pl
from jax.experimental.pallas import tpu as pltpu
from jax.experimental.pallas import tpu_sc as plsc


@functools.lru_cache(maxsize=None)
def _build_gather(B, D, chunk):
    info = plsc.get_sparse_core_info()
    NC, NS = info.num_cores, info.num_subcores
    NW = NC * NS
    assert B % (NW * chunk) == 0
    b_per_w = B // NW
    n_chunks = b_per_w // chunk
    mesh = plsc.VectorSubcoreMesh(core_axis_name="c", subcore_axis_name="s")

    nbuf = 6

    @functools.partial(
        pl.kernel,
        mesh=mesh,
        out_type=jax.ShapeDtypeStruct((B, D), jnp.float32),
        scratch_types=[
            pltpu.VMEM((n_chunks, chunk), jnp.int32),
        ]
        + [pltpu.VMEM((chunk, D), jnp.float32) for _ in range(nbuf)]
        + [pltpu.SemaphoreType.DMA for _ in range(2 * nbuf)],
    )
    def k(idx_hbm, table_hbm, out_hbm, idx_v, *rest):
        bufs = rest[:nbuf]
        gsems = rest[nbuf : 2 * nbuf]
        ssems = rest[2 * nbuf :]
        wid = lax.axis_index("s") * NC + lax.axis_index("c")
        pltpu.sync_copy(idx_hbm.at[wid], idx_v)
        gcp = [None] * nbuf
        scp = [None] * nbuf
        for step in range(n_chunks + nbuf - 1):
            slot = step % nbuf
            if step < n_chunks:
                if step >= nbuf:
                    scp[slot].wait()
                gcp[slot] = pltpu.async_copy(
                    table_hbm.at[idx_v.at[step]], bufs[slot], gsems[slot]
                )
            g = step - (nbuf - 1)
            if g >= 0:
                gslot = g % nbuf
                gcp[gslot].wait()
                scp[gslot] = pltpu.async_copy(
                    bufs[gslot],
                    out_hbm.at[pl.ds((g * NW + wid) * chunk, chunk)],
                    ssems[gslot],
                )
        for g in range(max(0, n_chunks - nbuf), n_chunks):
            scp[g % nbuf].wait()

    return k, NW, n_chunks, chunk


def kernel(positions, pe):
    Bb, S = positions.shape
    V, D = pe.shape
    B = Bb * S
    chunk = 16
    k, NW, n_chunks, chunk = _build_gather(B, D, chunk)
    idx = (
        positions.reshape(n_chunks, NW, chunk).transpose(1, 0, 2).astype(jnp.int32)
    )
    out = k(idx, pe)
    return out.reshape(Bb, S, D)


# speedup vs baseline: 1.0338x; 1.0080x over previous
import functools

import jax
import jax.numpy as jnp
from jax import lax
from jax.experimental import pallas as pl
from jax.experimental.pallas import tpu as pltpu
from jax.experimental.pallas import tpu_sc as plsc


@functools.lru_cache(maxsize=None)
def _build_gather(B, D, chunk):
    info = plsc.get_sparse_core_info()
    NC, NS = info.num_cores, info.num_subcores
    NW = NC * NS
    assert B % (NW * chunk) == 0
    b_per_w = B // NW
    n_chunks = b_per_w // chunk
    mesh = plsc.VectorSubcoreMesh(core_axis_name="c", subcore_axis_name="s")

    nbuf = 7

    @functools.partial(
        pl.kernel,
        mesh=mesh,
        out_type=jax.ShapeDtypeStruct((B, D), jnp.float32),
        scratch_types=[
            pltpu.VMEM((n_chunks, chunk), jnp.int32),
        ]
        + [pltpu.VMEM((chunk, D), jnp.float32) for _ in range(nbuf)]
        + [pltpu.SemaphoreType.DMA for _ in range(2 * nbuf)],
    )
    def k(idx_hbm, table_hbm, out_hbm, idx_v, *rest):
        bufs = rest[:nbuf]
        gsems = rest[nbuf : 2 * nbuf]
        ssems = rest[2 * nbuf :]
        wid = lax.axis_index("s") * NC + lax.axis_index("c")
        pltpu.sync_copy(idx_hbm.at[wid], idx_v)
        gcp = [None] * nbuf
        scp = [None] * nbuf
        for step in range(n_chunks + nbuf - 1):
            slot = step % nbuf
            if step < n_chunks:
                if step >= nbuf:
                    scp[slot].wait()
                gcp[slot] = pltpu.async_copy(
                    table_hbm.at[idx_v.at[step]], bufs[slot], gsems[slot]
                )
            g = step - (nbuf - 1)
            if g >= 0:
                gslot = g % nbuf
                gcp[gslot].wait()
                scp[gslot] = pltpu.async_copy(
                    bufs[gslot],
                    out_hbm.at[pl.ds((g * NW + wid) * chunk, chunk)],
                    ssems[gslot],
                )
        for g in range(max(0, n_chunks - nbuf), n_chunks):
            scp[g % nbuf].wait()

    return k, NW, n_chunks, chunk


def kernel(positions, pe):
    Bb, S = positions.shape
    V, D = pe.shape
    B = Bb * S
    chunk = 16
    k, NW, n_chunks, chunk = _build_gather(B, D, chunk)
    idx = (
        positions.reshape(n_chunks, NW, chunk).transpose(1, 0, 2).astype(jnp.int32)
    )
    out = k(idx, pe)
    return out.reshape(Bb, S, D)
